# Initial kernel scaffold; baseline (speedup 1.0000x reference)
import jax
import jax.numpy as jnp
from jax.experimental import pallas as pl


def kernel(teacher_top1_sim_pred, student_top1_sim_pred):
    raise NotImplementedError("write your pallas kernel here")


# speedup vs baseline: 1.7322x; 1.7322x over previous
import functools
import math

import jax
import jax.numpy as jnp
from jax.experimental import pallas as pl
from jax.experimental.pallas import tpu as pltpu

GAMMA_C = 1.0
EPS_C = 1e-07


def _roll_left(x, j):
    return jnp.concatenate([x[:, j:], x[:, :j]], axis=1)


def _roll_right(x, j):
    return jnp.concatenate([x[:, -j:], x[:, :-j]], axis=1)


def _listmle_body(t_ref, p_ref, out_ref):
    t = t_ref[...]
    p = p_ref[...]
    rows, n = t.shape

    mask = t == -1.0
    neg_inf = jnp.float32(-jnp.inf)
    pmasked = jnp.where(mask, neg_inf, p)
    mx = jnp.max(pmasked, axis=1, keepdims=True)
    pm = pmasked - mx
    sum_pm = jnp.sum(jnp.where(mask, 0.0, pm), axis=1)
    e = jnp.exp(pm)
    val = jnp.where(mask, -1.0, e)

    key = t
    lane = jax.lax.broadcasted_iota(jnp.int32, (rows, n), 1)

    logn = int(math.log2(n))
    for kk in range(1, logn + 1):
        k = 1 << kk
        for jj in range(kk - 1, -1, -1):
            j = 1 << jj
            lower = (lane & j) == 0
            up = (lane & k) == 0
            key_p = jnp.where(lower, _roll_left(key, j), _roll_right(key, j))
            val_p = jnp.where(lower, _roll_left(val, j), _roll_right(val, j))
            lo_v = jnp.where(lower, key, key_p)
            hi_v = jnp.where(lower, key_p, key)
            swap = (up & (lo_v > hi_v)) | (~up & (lo_v < hi_v))
            key = jnp.where(swap, key_p, key)
            val = jnp.where(swap, val_p, val)

    em = jnp.maximum(val, 0.0)
    c = em
    s = 1
    while s < n:
        c = c + jnp.where(lane >= s, _roll_right(c, s), 0.0)
        s *= 2

    log_term = jnp.where(val < 0.0, 0.0, jnp.log(c + EPS_C))
    row_loss = jnp.sum(log_term, axis=1) - sum_pm
    out_ref[0, 0] = GAMMA_C * jnp.mean(row_loss)


@functools.partial(jax.jit, static_argnames=("interpret",))
def _listmle_call(teacher, student, interpret=False):
    out = pl.pallas_call(
        _listmle_body,
        out_shape=jax.ShapeDtypeStruct((1, 1), jnp.float32),
        out_specs=pl.BlockSpec(memory_space=pltpu.SMEM),
        interpret=interpret,
    )(teacher, student)
    return out[0, 0]


def kernel(teacher_top1_sim_pred, student_top1_sim_pred):
    return _listmle_call(teacher_top1_sim_pred, student_top1_sim_pred)
